# Initial kernel scaffold; baseline (speedup 1.0000x reference)
import jax
import jax.numpy as jnp
from jax.experimental import pallas as pl


def kernel(x, edge_index, W1l, b1l, W1r, W2l, b2l, W2r):
    raise NotImplementedError("write your pallas kernel here")


# speedup vs baseline: 8.0890x; 8.0890x over previous
import functools
import jax
import jax.numpy as jnp
from jax import lax
from jax.experimental import pallas as pl
from jax.experimental.pallas import tpu as pltpu
from jax.experimental.pallas import tpu_sc as plsc

N_NODES = 10000
N_EDGES = 320000
D = 128

NC = 2
NS = 16
NW = NC * NS
E_PER_W = N_EDGES // NW
CHUNK = 200
N_ITERS = E_PER_W // CHUNK
NPAD = 10240
ROWS_PER_TILE = NPAD // NS
ZROWS = 128

_MESH = dict(core_axis_name="c", subcore_axis_name="s", num_cores=NC,
             num_subcores=NS)


def _segsum_body(table_hbm, src_hbm, dst_hbm, out_hbm,
                 acc, src_i, dst_i, rows_v, zrows, sem):
  cid = lax.axis_index("c")
  sid = lax.axis_index("s")
  wid = sid * NC + cid

  z16 = jnp.zeros((16,), jnp.float32)

  def zfill(i, _):
    r = i // (D // 16)
    c = (i % (D // 16)) * 16
    zrows[r, pl.ds(c, 16)] = z16
    return 0
  lax.fori_loop(0, ZROWS * (D // 16), zfill, 0)

  row0 = sid * ROWS_PER_TILE
  for k in range(ROWS_PER_TILE // ZROWS):
    pltpu.sync_copy(zrows, acc.at[pl.ds(row0 + k * ZROWS, ZROWS)])

  plsc.subcore_barrier()

  ebase = wid * E_PER_W
  def step(it, _):
    base = ebase + it * CHUNK
    pltpu.sync_copy(src_hbm.at[pl.ds(base, CHUNK)], src_i)
    pltpu.sync_copy(dst_hbm.at[pl.ds(base, CHUNK)], dst_i)
    pltpu.async_copy(table_hbm.at[src_i], rows_v, sem).wait()
    pltpu.sync_copy(rows_v, acc.at[dst_i], add=True)
    return 0
  lax.fori_loop(0, N_ITERS, step, 0)

  plsc.subcore_barrier()

  pltpu.sync_copy(acc.at[pl.ds(row0, ROWS_PER_TILE)],
                  out_hbm.at[cid].at[pl.ds(row0, ROWS_PER_TILE)])


_segsum = pl.kernel(
    _segsum_body,
    out_type=jax.ShapeDtypeStruct((NC, NPAD, D), jnp.float32),
    mesh=plsc.VectorSubcoreMesh(**_MESH),
    scratch_types=[
        pltpu.VMEM_SHARED((NPAD, D), jnp.float32),
        pltpu.VMEM((CHUNK,), jnp.int32),
        pltpu.VMEM((CHUNK,), jnp.int32),
        pltpu.VMEM((CHUNK, D), jnp.float32),
        pltpu.VMEM((ZROWS, D), jnp.float32),
        pltpu.SemaphoreType.DMA,
    ])

CCHUNK = 2000
C_ITERS = E_PER_W // CCHUNK


def _cnt_body(dst_hbm, out_hbm, acc_c, dst_i, ones_v, zcnt):
  cid = lax.axis_index("c")
  sid = lax.axis_index("s")
  wid = sid * NC + cid

  z16 = jnp.zeros((16,), jnp.float32)
  one16 = jnp.ones((16,), jnp.float32)

  def zfill(i, _):
    zcnt[pl.ds(i * 16, 16)] = z16
    return 0
  lax.fori_loop(0, ROWS_PER_TILE // 16, zfill, 0)

  def ofill(i, _):
    ones_v[pl.ds(i * 16, 16)] = one16
    return 0
  lax.fori_loop(0, CCHUNK // 16, ofill, 0)

  row0 = sid * ROWS_PER_TILE
  pltpu.sync_copy(zcnt, acc_c.at[pl.ds(row0, ROWS_PER_TILE)])
  plsc.subcore_barrier()

  ebase = wid * E_PER_W
  def step(it, _):
    base = ebase + it * CCHUNK
    pltpu.sync_copy(dst_hbm.at[pl.ds(base, CCHUNK)], dst_i)
    pltpu.sync_copy(ones_v, acc_c.at[dst_i], add=True)
    return 0
  lax.fori_loop(0, C_ITERS, step, 0)

  plsc.subcore_barrier()
  pltpu.sync_copy(acc_c.at[pl.ds(row0, ROWS_PER_TILE)],
                  out_hbm.at[cid].at[pl.ds(row0, ROWS_PER_TILE)])


_cnt = pl.kernel(
    _cnt_body,
    out_type=jax.ShapeDtypeStruct((NC, NPAD), jnp.float32),
    mesh=plsc.VectorSubcoreMesh(**_MESH),
    scratch_types=[
        pltpu.VMEM_SHARED((NPAD,), jnp.float32),
        pltpu.VMEM((CCHUNK,), jnp.int32),
        pltpu.VMEM((CCHUNK,), jnp.float32),
        pltpu.VMEM((ROWS_PER_TILE,), jnp.float32),
    ])

assert CCHUNK % 16 == 0 and ROWS_PER_TILE % 16 == 0

ROW_BLK = 1024
N_BLKS = NPAD // ROW_BLK


def _dense_body(relu, p_ref, pc_ref, x_ref, wl_ref, b_ref, wr_ref, o_ref):
  agg = p_ref[0] + p_ref[1]
  cnt = pc_ref[0] + pc_ref[1]
  mean = agg / jnp.maximum(cnt, 1.0)
  dn = (((1,), (1,)), ((), ()))
  out = (lax.dot_general(mean, wl_ref[...], dn,
                         preferred_element_type=jnp.float32,
                         precision=lax.Precision.HIGHEST)
         + b_ref[...]
         + lax.dot_general(x_ref[...], wr_ref[...], dn,
                           preferred_element_type=jnp.float32,
                           precision=lax.Precision.HIGHEST))
  o_ref[...] = jnp.maximum(out, 0.0) if relu else out


def _make_dense(relu):
  return pl.pallas_call(
      functools.partial(_dense_body, relu),
      grid=(N_BLKS,),
      in_specs=[
          pl.BlockSpec((NC, ROW_BLK, D), lambda i: (0, i, 0)),
          pl.BlockSpec((NC, ROW_BLK, 1), lambda i: (0, i, 0)),
          pl.BlockSpec((ROW_BLK, D), lambda i: (i, 0)),
          pl.BlockSpec((D, D), lambda i: (0, 0)),
          pl.BlockSpec((1, D), lambda i: (0, 0)),
          pl.BlockSpec((D, D), lambda i: (0, 0)),
      ],
      out_specs=pl.BlockSpec((ROW_BLK, D), lambda i: (i, 0)),
      out_shape=jax.ShapeDtypeStruct((N_NODES, D), jnp.float32),
  )


_dense_relu = _make_dense(True)
_dense_lin = _make_dense(False)


def kernel(x, edge_index, W1l, b1l, W1r, W2l, b2l, W2r):
  src = edge_index[0]
  dst = edge_index[1]
  b1 = b1l.reshape(1, D)
  b2 = b2l.reshape(1, D)

  pc = _cnt(dst).reshape(NC, NPAD, 1)
  p1 = _segsum(x, src, dst)
  h = _dense_relu(p1, pc, x, W1l, b1, W1r)
  p2 = _segsum(h, src, dst)
  out = _dense_lin(p2, pc, h, W2l, b2, W2r)
  return out
